# baseline (speedup 1.0000x reference)
import jax
import jax.numpy as jnp
from jax import lax
from jax.experimental import pallas as pl
from jax.experimental.pallas import tpu as pltpu
from jax.experimental.pallas import tpu_sc as plsc

_BATCH = 1048576
_NBITS = 32

_SC_ROWS = 294912
_TC_ROWS = _BATCH - _SC_ROWS

_ROWS_PER_BLOCK = 16384
_TC_GRID = _TC_ROWS // _ROWS_PER_BLOCK
_CHUNK = 512

_NW = 32
_RPW = _SC_ROWS // _NW
_SC_UNROLL = 4
_SC_GROUP = 16 * _SC_UNROLL
_SC_STEPS = _RPW // _SC_GROUP

_ROT_A = (13, 15, 26, 6)
_ROT_B = (17, 29, 16, 24)
_K0 = 0
_K1 = 42
_K2 = _K0 ^ _K1 ^ 0x1BD11BDA
_KS = (_K0, _K1, _K2)


def _threefry_bits(lo):
    x1i = lo + jnp.uint32(_K1)
    x0 = x1i
    x1 = ((x1i << jnp.uint32(_ROT_A[0])) |
          (x1i >> jnp.uint32(32 - _ROT_A[0]))) ^ x1i
    for i in range(5):
        for r in (_ROT_A[1:] if i == 0 else (_ROT_A if i % 2 == 0 else _ROT_B)):
            x0 = x0 + x1
            x1 = (x1 << jnp.uint32(r)) | (x1 >> jnp.uint32(32 - r))
            x1 = x1 ^ x0
        x0 = x0 + jnp.uint32(_KS[(i + 1) % 3])
        x1 = x1 + jnp.uint32((_KS[(i + 2) % 3] + i + 1) & 0xFFFFFFFF)
    return x0 ^ x1


def _tc_kernel(thresh_ref, out_ref):
    pid = pl.program_id(0)
    r0 = (pid * _ROWS_PER_BLOCK).astype(jnp.uint32)
    thresh = thresh_ref[0, 0]

    j = lax.broadcasted_iota(jnp.uint32, (_NBITS, _CHUNK), 0)
    c = lax.broadcasted_iota(jnp.uint32, (_NBITS, _CHUNK), 1)
    lin = c * jnp.uint32(_NBITS) + j
    pow2j = lax.bitcast_convert_type(jnp.uint32(1) << j, jnp.int32)
    zero = jnp.zeros((_NBITS, _CHUNK), jnp.int32)

    for chunk in range(_ROWS_PER_BLOCK // _CHUNK):
        base = (r0 + jnp.uint32(chunk * _CHUNK)) * jnp.uint32(_NBITS)
        bits = _threefry_bits(lin + base)
        shifted = jnp.where((bits >> jnp.uint32(9)) < thresh, pow2j, zero)
        packed = jnp.sum(shifted, axis=0, dtype=jnp.int32)
        out_ref[0, 0, chunk * _CHUNK:(chunk + 1) * _CHUNK] = (
            lax.bitcast_convert_type(packed, jnp.uint32))


def _tc_call(thresh_u32):
    return pl.pallas_call(
        _tc_kernel,
        grid=(_TC_GRID,),
        in_specs=[pl.BlockSpec((1, 1), lambda i: (0, 0))],
        out_specs=pl.BlockSpec((1, 1, _ROWS_PER_BLOCK), lambda i: (i, 0, 0)),
        out_shape=jax.ShapeDtypeStruct(
            (_TC_GRID, 1, _ROWS_PER_BLOCK), jnp.uint32),
    )(thresh_u32.reshape(1, 1)).reshape(_TC_ROWS)


def _sc_body(thresh_hbm, out_hbm, thresh_v, buf):
    core = lax.axis_index("c")
    sub = lax.axis_index("s")
    wid = sub * 2 + core
    pltpu.sync_copy(thresh_hbm, thresh_v)
    thresh = thresh_v[...]

    lanes = lax.broadcasted_iota(jnp.int32, (16,), 0).astype(jnp.uint32)
    lin_lane = lanes * jnp.uint32(_NBITS)
    base_w = (jnp.uint32(_TC_ROWS) +
              wid.astype(jnp.uint32) * jnp.uint32(_RPW)) * jnp.uint32(_NBITS)
    zero = jnp.zeros((16,), jnp.uint32)
    one = jnp.full((16,), 1, jnp.uint32)

    def g_body(g, carry):
        b0 = base_w + g.astype(jnp.uint32) * jnp.uint32(_SC_GROUP * _NBITS)
        lins = [lin_lane + (b0 + jnp.uint32(k * 16 * _NBITS))
                for k in range(_SC_UNROLL)]

        def j_body(jj, accs):
            ju = (jj + jj).astype(jnp.uint32)
            bit0 = one << ju
            bit1 = bit0 + bit0
            out = []
            for lin, acc in zip(lins, accs):
                bits0 = _threefry_bits(lin + ju)
                bits1 = _threefry_bits(lin + (ju + jnp.uint32(1)))
                sel0 = jnp.where((bits0 >> jnp.uint32(9)) < thresh, bit0, zero)
                sel1 = jnp.where((bits1 >> jnp.uint32(9)) < thresh, bit1, zero)
                out.append(acc | sel0 | sel1)
            return tuple(out)

        accs = lax.fori_loop(0, _NBITS // 2, j_body,
                             tuple(zero for _ in range(_SC_UNROLL)))
        for k in range(_SC_UNROLL):
            buf[pl.ds(g * _SC_GROUP + k * 16, 16)] = accs[k]
        return carry

    lax.fori_loop(0, _SC_STEPS, g_body, 0)
    pltpu.sync_copy(buf, out_hbm.at[pl.ds(wid * _RPW, _RPW)])


def _sc_call(thresh_u32):
    mesh = plsc.VectorSubcoreMesh(core_axis_name="c", subcore_axis_name="s")
    run = pl.kernel(
        _sc_body,
        out_type=jax.ShapeDtypeStruct((_SC_ROWS,), jnp.uint32),
        mesh=mesh,
        scratch_types=[
            pltpu.VMEM((16,), jnp.uint32),
            pltpu.VMEM((_RPW,), jnp.uint32),
        ],
    )
    return run(jnp.full((16,), thresh_u32, jnp.uint32))


def kernel(inputs, probs):
    del inputs
    p = jnp.asarray(probs, dtype=jnp.float32)
    t = jnp.clip(jnp.ceil(p * jnp.float32(8388608.0)), 0.0, 8388608.0)
    thresh = t.astype(jnp.uint32)
    tc_out = _tc_call(thresh)
    sc_out = _sc_call(thresh)
    return jnp.concatenate([tc_out, sc_out])


# speedup vs baseline: 1.0478x; 1.0478x over previous
import jax
import jax.numpy as jnp
from jax import lax
from jax.experimental import pallas as pl
from jax.experimental.pallas import tpu as pltpu
from jax.experimental.pallas import tpu_sc as plsc

_BATCH = 1048576
_NBITS = 32

_SC_ROWS = 278528
_TC_ROWS = _BATCH - _SC_ROWS

_ROWS_PER_BLOCK = 16384
_TC_GRID = _TC_ROWS // _ROWS_PER_BLOCK
_CHUNK = 512

_NW = 32
_RPW = _SC_ROWS // _NW
_SC_UNROLL = 4
_SC_GROUP = 16 * _SC_UNROLL
_SC_STEPS = _RPW // _SC_GROUP

_ROT_A = (13, 15, 26, 6)
_ROT_B = (17, 29, 16, 24)
_K0 = 0
_K1 = 42
_K2 = _K0 ^ _K1 ^ 0x1BD11BDA
_KS = (_K0, _K1, _K2)


def _threefry_bits(lo):
    x1i = lo + jnp.uint32(_K1)
    x0 = x1i
    x1 = ((x1i << jnp.uint32(_ROT_A[0])) |
          (x1i >> jnp.uint32(32 - _ROT_A[0]))) ^ x1i
    for i in range(5):
        for r in (_ROT_A[1:] if i == 0 else (_ROT_A if i % 2 == 0 else _ROT_B)):
            x0 = x0 + x1
            x1 = (x1 << jnp.uint32(r)) | (x1 >> jnp.uint32(32 - r))
            x1 = x1 ^ x0
        x0 = x0 + jnp.uint32(_KS[(i + 1) % 3])
        x1 = x1 + jnp.uint32((_KS[(i + 2) % 3] + i + 1) & 0xFFFFFFFF)
    return x0 ^ x1


def _tc_kernel(thresh_ref, out_ref):
    pid = pl.program_id(0)
    r0 = (pid * _ROWS_PER_BLOCK).astype(jnp.uint32)
    thresh = thresh_ref[0, 0]

    j = lax.broadcasted_iota(jnp.uint32, (_NBITS, _CHUNK), 0)
    c = lax.broadcasted_iota(jnp.uint32, (_NBITS, _CHUNK), 1)
    lin = c * jnp.uint32(_NBITS) + j
    pow2j = lax.bitcast_convert_type(jnp.uint32(1) << j, jnp.int32)
    zero = jnp.zeros((_NBITS, _CHUNK), jnp.int32)

    for chunk in range(_ROWS_PER_BLOCK // _CHUNK):
        base = (r0 + jnp.uint32(chunk * _CHUNK)) * jnp.uint32(_NBITS)
        bits = _threefry_bits(lin + base)
        shifted = jnp.where((bits >> jnp.uint32(9)) < thresh, pow2j, zero)
        packed = jnp.sum(shifted, axis=0, dtype=jnp.int32)
        out_ref[0, 0, chunk * _CHUNK:(chunk + 1) * _CHUNK] = (
            lax.bitcast_convert_type(packed, jnp.uint32))


def _tc_call(thresh_u32):
    return pl.pallas_call(
        _tc_kernel,
        grid=(_TC_GRID,),
        in_specs=[pl.BlockSpec((1, 1), lambda i: (0, 0))],
        out_specs=pl.BlockSpec((1, 1, _ROWS_PER_BLOCK), lambda i: (i, 0, 0)),
        out_shape=jax.ShapeDtypeStruct(
            (_TC_GRID, 1, _ROWS_PER_BLOCK), jnp.uint32),
    )(thresh_u32.reshape(1, 1)).reshape(_TC_ROWS)


def _sc_body(thresh_hbm, out_hbm, thresh_v, buf):
    core = lax.axis_index("c")
    sub = lax.axis_index("s")
    wid = sub * 2 + core
    pltpu.sync_copy(thresh_hbm, thresh_v)
    thresh = thresh_v[...]

    lanes = lax.broadcasted_iota(jnp.int32, (16,), 0).astype(jnp.uint32)
    lin_lane = lanes * jnp.uint32(_NBITS)
    base_w = (jnp.uint32(_TC_ROWS) +
              wid.astype(jnp.uint32) * jnp.uint32(_RPW)) * jnp.uint32(_NBITS)
    zero = jnp.zeros((16,), jnp.uint32)
    one = jnp.full((16,), 1, jnp.uint32)

    def g_body(g, carry):
        b0 = base_w + g.astype(jnp.uint32) * jnp.uint32(_SC_GROUP * _NBITS)
        lins = [lin_lane + (b0 + jnp.uint32(k * 16 * _NBITS))
                for k in range(_SC_UNROLL)]

        def j_body(j, accs):
            ju = j.astype(jnp.uint32)
            bit = one << ju
            out = []
            for lin, acc in zip(lins, accs):
                bits = _threefry_bits(lin + ju)
                mask = (bits >> jnp.uint32(9)) < thresh
                out.append(acc | jnp.where(mask, bit, zero))
            return tuple(out)

        accs = lax.fori_loop(0, _NBITS, j_body,
                             tuple(zero for _ in range(_SC_UNROLL)))
        for k in range(_SC_UNROLL):
            buf[pl.ds(g * _SC_GROUP + k * 16, 16)] = accs[k]
        return carry

    lax.fori_loop(0, _SC_STEPS, g_body, 0)
    pltpu.sync_copy(buf, out_hbm.at[pl.ds(wid * _RPW, _RPW)])


def _sc_call(thresh_u32):
    mesh = plsc.VectorSubcoreMesh(core_axis_name="c", subcore_axis_name="s")
    run = pl.kernel(
        _sc_body,
        out_type=jax.ShapeDtypeStruct((_SC_ROWS,), jnp.uint32),
        mesh=mesh,
        scratch_types=[
            pltpu.VMEM((16,), jnp.uint32),
            pltpu.VMEM((_RPW,), jnp.uint32),
        ],
    )
    return run(jnp.full((16,), thresh_u32, jnp.uint32))


def kernel(inputs, probs):
    del inputs
    p = jnp.asarray(probs, dtype=jnp.float32)
    t = jnp.clip(jnp.ceil(p * jnp.float32(8388608.0)), 0.0, 8388608.0)
    thresh = t.astype(jnp.uint32)
    tc_out = _tc_call(thresh)
    sc_out = _sc_call(thresh)
    return jnp.concatenate([tc_out, sc_out])


# speedup vs baseline: 1.0513x; 1.0034x over previous
import jax
import jax.numpy as jnp
from jax import lax
from jax.experimental import pallas as pl
from jax.experimental.pallas import tpu as pltpu
from jax.experimental.pallas import tpu_sc as plsc

_BATCH = 1048576
_NBITS = 32

_SC_ROWS = 278528
_TC_ROWS = _BATCH - _SC_ROWS

_ROWS_PER_BLOCK = 16384
_TC_GRID = _TC_ROWS // _ROWS_PER_BLOCK
_CHUNK = 512

_NW = 32
_RPW = _SC_ROWS // _NW
_SC_UNROLL = 4
_SC_GROUP = 16 * _SC_UNROLL
_SC_STEPS = _RPW // _SC_GROUP

_ROT_A = (13, 15, 26, 6)
_ROT_B = (17, 29, 16, 24)
_K0 = 0
_K1 = 42
_K2 = _K0 ^ _K1 ^ 0x1BD11BDA
_KS = (_K0, _K1, _K2)


def _threefry_bits(lo):
    x1i = lo + jnp.uint32(_K1)
    x0 = x1i
    x1 = ((x1i << jnp.uint32(_ROT_A[0])) |
          (x1i >> jnp.uint32(32 - _ROT_A[0]))) ^ x1i
    for i in range(5):
        for r in (_ROT_A[1:] if i == 0 else (_ROT_A if i % 2 == 0 else _ROT_B)):
            x0 = x0 + x1
            x1 = (x1 << jnp.uint32(r)) | (x1 >> jnp.uint32(32 - r))
            x1 = x1 ^ x0
        x0 = x0 + jnp.uint32(_KS[(i + 1) % 3])
        x1 = x1 + jnp.uint32((_KS[(i + 2) % 3] + i + 1) & 0xFFFFFFFF)
    return x0 ^ x1


def _int_threshold(p):
    x = jnp.clip(p * jnp.float32(8388608.0), jnp.float32(0.0),
                 jnp.float32(8388608.0))
    xi = x.astype(jnp.int32)
    xi = xi + (xi.astype(jnp.float32) < x).astype(jnp.int32)
    return xi.astype(jnp.uint32)


def _tc_kernel(probs_ref, out_ref):
    pid = pl.program_id(0)
    r0 = (pid * _ROWS_PER_BLOCK).astype(jnp.uint32)
    thresh = _int_threshold(probs_ref[0, 0])

    j = lax.broadcasted_iota(jnp.uint32, (_NBITS, _CHUNK), 0)
    c = lax.broadcasted_iota(jnp.uint32, (_NBITS, _CHUNK), 1)
    lin = c * jnp.uint32(_NBITS) + j
    pow2j = lax.bitcast_convert_type(jnp.uint32(1) << j, jnp.int32)
    zero = jnp.zeros((_NBITS, _CHUNK), jnp.int32)

    for chunk in range(_ROWS_PER_BLOCK // _CHUNK):
        base = (r0 + jnp.uint32(chunk * _CHUNK)) * jnp.uint32(_NBITS)
        bits = _threefry_bits(lin + base)
        shifted = jnp.where((bits >> jnp.uint32(9)) < thresh, pow2j, zero)
        packed = jnp.sum(shifted, axis=0, dtype=jnp.int32)
        out_ref[0, 0, chunk * _CHUNK:(chunk + 1) * _CHUNK] = (
            lax.bitcast_convert_type(packed, jnp.uint32))


def _tc_call(probs_f32):
    return pl.pallas_call(
        _tc_kernel,
        grid=(_TC_GRID,),
        in_specs=[pl.BlockSpec((1, 1), lambda i: (0, 0))],
        out_specs=pl.BlockSpec((1, 1, _ROWS_PER_BLOCK), lambda i: (i, 0, 0)),
        out_shape=jax.ShapeDtypeStruct(
            (_TC_GRID, 1, _ROWS_PER_BLOCK), jnp.uint32),
    )(probs_f32.reshape(1, 1)).reshape(_TC_ROWS)


def _sc_body(thresh_hbm, out_hbm, thresh_v, buf):
    core = lax.axis_index("c")
    sub = lax.axis_index("s")
    wid = sub * 2 + core
    pltpu.sync_copy(thresh_hbm, thresh_v)
    thresh = thresh_v[...]

    lanes = lax.broadcasted_iota(jnp.int32, (16,), 0).astype(jnp.uint32)
    lin_lane = lanes * jnp.uint32(_NBITS)
    base_w = (jnp.uint32(_TC_ROWS) +
              wid.astype(jnp.uint32) * jnp.uint32(_RPW)) * jnp.uint32(_NBITS)
    zero = jnp.zeros((16,), jnp.uint32)
    one = jnp.full((16,), 1, jnp.uint32)

    def g_body(g, carry):
        b0 = base_w + g.astype(jnp.uint32) * jnp.uint32(_SC_GROUP * _NBITS)
        lins = [lin_lane + (b0 + jnp.uint32(k * 16 * _NBITS))
                for k in range(_SC_UNROLL)]

        def j_body(j, accs):
            ju = j.astype(jnp.uint32)
            bit = one << ju
            out = []
            for lin, acc in zip(lins, accs):
                bits = _threefry_bits(lin + ju)
                mask = (bits >> jnp.uint32(9)) < thresh
                out.append(acc | jnp.where(mask, bit, zero))
            return tuple(out)

        accs = lax.fori_loop(0, _NBITS, j_body,
                             tuple(zero for _ in range(_SC_UNROLL)))
        for k in range(_SC_UNROLL):
            buf[pl.ds(g * _SC_GROUP + k * 16, 16)] = accs[k]
        return carry

    lax.fori_loop(0, _SC_STEPS, g_body, 0)
    pltpu.sync_copy(buf, out_hbm.at[pl.ds(wid * _RPW, _RPW)])


def _sc_call(probs_f32):
    mesh = plsc.VectorSubcoreMesh(core_axis_name="c", subcore_axis_name="s")
    run = pl.kernel(
        _sc_body,
        out_type=jax.ShapeDtypeStruct((_SC_ROWS,), jnp.uint32),
        mesh=mesh,
        scratch_types=[
            pltpu.VMEM((16,), jnp.uint32),
            pltpu.VMEM((_RPW,), jnp.uint32),
        ],
    )
    return run(jnp.full((16,), _int_threshold(probs_f32), jnp.uint32))


def kernel(inputs, probs):
    del inputs
    p = jnp.asarray(probs, dtype=jnp.float32)
    tc_out = _tc_call(p)
    sc_out = _sc_call(p)
    return jnp.concatenate([tc_out, sc_out])
